# baseline (speedup 1.0000x reference)
import jax
import jax.numpy as jnp
from jax.experimental import pallas as pl
from jax.experimental.pallas import tpu as pltpu

_N = 10000
_F = 128
_BN_EPS = 1e-5

_ROWS = 400
_NSTREAM = _N // _ROWS
_ROWS2 = 1000
_NEPI = _N // _ROWS2


def _fused_kernel(adj_ref, h_ref, w1_ref, b1_ref, eps_ref, w2_ref, b2_ref,
                  g_ref, be_ref, o_ref, h2_ref, s_ref, q_ref):
    i = pl.program_id(0)

    @pl.when(i < _NSTREAM)
    def _stream():
        a = adj_ref[...]
        raw = jnp.dot(a, h_ref[...], preferred_element_type=jnp.float32)
        deg = jnp.sum(a, axis=1, keepdims=True)
        pooled = jnp.dot(raw, w1_ref[...], preferred_element_type=jnp.float32)
        h1t = (
            jnp.dot(h_ref[pl.ds(i * _ROWS, _ROWS), :], w1_ref[...],
                    preferred_element_type=jnp.float32)
            + b1_ref[...]
        )
        h2 = (pooled / deg + b1_ref[...]) + eps_ref[0, 0] * h1t
        h2_ref[pl.ds(i * _ROWS, _ROWS), :] = h2
        s = jnp.sum(h2, axis=0, keepdims=True)
        q = jnp.sum(h2 * h2, axis=0, keepdims=True)

        @pl.when(i == 0)
        def _init():
            s_ref[...] = s
            q_ref[...] = q

        @pl.when(i > 0)
        def _acc():
            s_ref[...] += s
            q_ref[...] += q

    @pl.when(i >= _NSTREAM)
    def _epilogue():
        j = i - _NSTREAM
        mean = s_ref[...] * (1.0 / _N)
        var = q_ref[...] * (1.0 / _N) - mean * mean
        scale = jax.lax.rsqrt(var + _BN_EPS) * g_ref[...]
        h2t = h2_ref[pl.ds(j * _ROWS2, _ROWS2), :]
        hn = (h2t - mean) * scale + be_ref[...]
        o_ref[...] = (
            jnp.dot(hn, w2_ref[...], preferred_element_type=jnp.float32)
            + b2_ref[...]
        )


def kernel(h, adj, W1, b1, W2, b2, gamma, beta, eps1):
    f32 = jnp.float32
    w1t = W1.T
    w2t = W2.T
    b1r = b1.reshape(1, _F)
    b2r = b2.reshape(1, _F)
    gr = gamma.reshape(1, _F)
    ber = beta.reshape(1, _F)
    epsr = eps1.reshape(1, 1)

    const = lambda i: (0, 0)

    out = pl.pallas_call(
        _fused_kernel,
        grid=(_NSTREAM + _NEPI,),
        in_specs=[
            pl.BlockSpec((_ROWS, _N), lambda i: (jnp.minimum(i, _NSTREAM - 1), 0)),
            pl.BlockSpec((_N, _F), const),
            pl.BlockSpec((_F, _F), const),
            pl.BlockSpec((1, _F), const),
            pl.BlockSpec((1, 1), const),
            pl.BlockSpec((_F, _F), const),
            pl.BlockSpec((1, _F), const),
            pl.BlockSpec((1, _F), const),
            pl.BlockSpec((1, _F), const),
        ],
        out_specs=pl.BlockSpec(
            (_ROWS2, _F), lambda i: (jnp.maximum(i - _NSTREAM, 0), 0)
        ),
        out_shape=jax.ShapeDtypeStruct((_N, _F), f32),
        scratch_shapes=[
            pltpu.VMEM((_N, _F), f32),
            pltpu.VMEM((1, _F), f32),
            pltpu.VMEM((1, _F), f32),
        ],
        compiler_params=pltpu.CompilerParams(
            vmem_limit_bytes=62 * 1024 * 1024,
        ),
    )(adj, h, w1t, b1r, epsr, w2t, b2r, gr, ber)

    return out


# speedup vs baseline: 1.0105x; 1.0105x over previous
import jax
import jax.numpy as jnp
from jax.experimental import pallas as pl
from jax.experimental.pallas import tpu as pltpu

_N = 10000
_F = 128
_BN_EPS = 1e-5

_ROWS = 400
_NSTREAM = _N // _ROWS
_ROWS2 = 2000
_NEPI = _N // _ROWS2


def _fused_kernel(adj_ref, h_ref, w1_ref, b1_ref, eps_ref, w2_ref, b2_ref,
                  g_ref, be_ref, o_ref, h1_ref, h2_ref, s_ref, q_ref):
    i = pl.program_id(0)

    @pl.when(i == 0)
    def _compute_h1():
        h1_ref[...] = (
            jnp.dot(h_ref[...], w1_ref[...], preferred_element_type=jnp.float32)
            + b1_ref[...]
        )

    @pl.when(i < _NSTREAM)
    def _stream():
        a = adj_ref[...]
        pooled = jnp.dot(a, h1_ref[...], preferred_element_type=jnp.float32)
        deg = jnp.sum(a, axis=1, keepdims=True)
        h1t = h1_ref[pl.ds(i * _ROWS, _ROWS), :]
        h2 = pooled / deg + eps_ref[0, 0] * h1t
        h2_ref[pl.ds(i * _ROWS, _ROWS), :] = h2
        s = jnp.sum(h2, axis=0, keepdims=True)
        q = jnp.sum(h2 * h2, axis=0, keepdims=True)

        @pl.when(i == 0)
        def _init():
            s_ref[...] = s
            q_ref[...] = q

        @pl.when(i > 0)
        def _acc():
            s_ref[...] += s
            q_ref[...] += q

    @pl.when(i >= _NSTREAM)
    def _epilogue():
        j = i - _NSTREAM
        mean = s_ref[...] * (1.0 / _N)
        var = q_ref[...] * (1.0 / _N) - mean * mean
        scale = jax.lax.rsqrt(var + _BN_EPS) * g_ref[...]
        h2t = h2_ref[pl.ds(j * _ROWS2, _ROWS2), :]
        hn = (h2t - mean) * scale + be_ref[...]
        o_ref[...] = (
            jnp.dot(hn, w2_ref[...], preferred_element_type=jnp.float32)
            + b2_ref[...]
        )


def kernel(h, adj, W1, b1, W2, b2, gamma, beta, eps1):
    f32 = jnp.float32
    w1t = W1.T
    w2t = W2.T
    b1r = b1.reshape(1, _F)
    b2r = b2.reshape(1, _F)
    gr = gamma.reshape(1, _F)
    ber = beta.reshape(1, _F)
    epsr = eps1.reshape(1, 1)

    const = lambda i: (0, 0)

    out = pl.pallas_call(
        _fused_kernel,
        grid=(_NSTREAM + _NEPI,),
        in_specs=[
            pl.BlockSpec((_ROWS, _N), lambda i: (jnp.minimum(i, _NSTREAM - 1), 0)),
            pl.BlockSpec((_N, _F), const),
            pl.BlockSpec((_F, _F), const),
            pl.BlockSpec((1, _F), const),
            pl.BlockSpec((1, 1), const),
            pl.BlockSpec((_F, _F), const),
            pl.BlockSpec((1, _F), const),
            pl.BlockSpec((1, _F), const),
            pl.BlockSpec((1, _F), const),
        ],
        out_specs=pl.BlockSpec(
            (_ROWS2, _F), lambda i: (jnp.maximum(i - _NSTREAM, 0), 0)
        ),
        out_shape=jax.ShapeDtypeStruct((_N, _F), f32),
        scratch_shapes=[
            pltpu.VMEM((_N, _F), f32),
            pltpu.VMEM((_N, _F), f32),
            pltpu.VMEM((1, _F), f32),
            pltpu.VMEM((1, _F), f32),
        ],
        compiler_params=pltpu.CompilerParams(
            vmem_limit_bytes=62 * 1024 * 1024,
        ),
    )(adj, h, w1t, b1r, epsr, w2t, b2r, gr, ber)

    return out


# speedup vs baseline: 1.0207x; 1.0101x over previous
import jax
import jax.numpy as jnp
from jax.experimental import pallas as pl
from jax.experimental.pallas import tpu as pltpu

_N = 10000
_F = 128
_BN_EPS = 1e-5

_ROWS = 400
_NSTREAM = _N // _ROWS
_ROWS2 = 2000
_NEPI = _N // _ROWS2


def _fused_kernel(adjt_ref, adjb_ref, h_ref, w1_ref, b1_ref, eps_ref,
                  w2_ref, b2_ref, g_ref, be_ref, o_ref,
                  h1_ref, h2_ref, s_ref, q_ref):
    i = pl.program_id(0)

    @pl.when(i == 0)
    def _compute_h1():
        h1_ref[...] = (
            jnp.dot(h_ref[...], w1_ref[...], preferred_element_type=jnp.float32)
            + b1_ref[...]
        )

    @pl.when(i < _NSTREAM)
    def _stream():
        a = jnp.concatenate([adjt_ref[...], adjb_ref[...]], axis=0)
        pooled = jnp.dot(a, h1_ref[...], preferred_element_type=jnp.float32)
        deg = jnp.sum(a, axis=1, keepdims=True)
        h1t = h1_ref[pl.ds(i * _ROWS, _ROWS), :]
        h2 = pooled / deg + eps_ref[0, 0] * h1t
        h2_ref[pl.ds(i * _ROWS, _ROWS), :] = h2
        s = jnp.sum(h2, axis=0, keepdims=True)
        q = jnp.sum(h2 * h2, axis=0, keepdims=True)

        @pl.when(i == 0)
        def _init():
            s_ref[...] = s
            q_ref[...] = q

        @pl.when(i > 0)
        def _acc():
            s_ref[...] += s
            q_ref[...] += q

    @pl.when(i >= _NSTREAM)
    def _epilogue():
        j = i - _NSTREAM
        mean = s_ref[...] * (1.0 / _N)
        var = q_ref[...] * (1.0 / _N) - mean * mean
        scale = jax.lax.rsqrt(var + _BN_EPS) * g_ref[...]
        h2t = h2_ref[pl.ds(j * _ROWS2, _ROWS2), :]
        hn = (h2t - mean) * scale + be_ref[...]
        o_ref[...] = (
            jnp.dot(hn, w2_ref[...], preferred_element_type=jnp.float32)
            + b2_ref[...]
        )


def kernel(h, adj, W1, b1, W2, b2, gamma, beta, eps1):
    f32 = jnp.float32
    w1t = W1.T
    w2t = W2.T
    b1r = b1.reshape(1, _F)
    b2r = b2.reshape(1, _F)
    gr = gamma.reshape(1, _F)
    ber = beta.reshape(1, _F)
    epsr = eps1.reshape(1, 1)

    const = lambda i: (0, 0)

    out = pl.pallas_call(
        _fused_kernel,
        grid=(_NSTREAM + _NEPI,),
        in_specs=[
            pl.BlockSpec(
                (_ROWS // 2, _N),
                lambda i: (2 * jnp.minimum(i, _NSTREAM - 1), 0),
            ),
            pl.BlockSpec(
                (_ROWS // 2, _N),
                lambda i: (2 * jnp.minimum(i, _NSTREAM - 1) + 1, 0),
            ),
            pl.BlockSpec((_N, _F), const),
            pl.BlockSpec((_F, _F), const),
            pl.BlockSpec((1, _F), const),
            pl.BlockSpec((1, 1), const),
            pl.BlockSpec((_F, _F), const),
            pl.BlockSpec((1, _F), const),
            pl.BlockSpec((1, _F), const),
            pl.BlockSpec((1, _F), const),
        ],
        out_specs=pl.BlockSpec(
            (_ROWS2, _F), lambda i: (jnp.maximum(i - _NSTREAM, 0), 0)
        ),
        out_shape=jax.ShapeDtypeStruct((_N, _F), f32),
        scratch_shapes=[
            pltpu.VMEM((_N, _F), f32),
            pltpu.VMEM((_N, _F), f32),
            pltpu.VMEM((1, _F), f32),
            pltpu.VMEM((1, _F), f32),
        ],
        compiler_params=pltpu.CompilerParams(
            vmem_limit_bytes=62 * 1024 * 1024,
        ),
    )(adj, adj, h, w1t, b1r, epsr, w2t, b2r, gr, ber)

    return out
